# baseline (speedup 1.0000x reference)
import jax
import jax.numpy as jnp
from jax.experimental import pallas as pl
from jax.experimental.pallas import tpu as pltpu

_DN = (((1,), (0,)), ((), ()))
_SEQ = 512
_NRES = 4096
_NIN = 256


def _proj_kernel(x_ref, winT_ref, u_ref):
    u_ref[0] = jax.lax.dot_general(x_ref[0], winT_ref[...], _DN,
                                   preferred_element_type=jnp.float32)


def _esn_kernel(u_ref, wresT_ref, o_ref, state, part):
    t = pl.program_id(0)

    @pl.when(t == 0)
    def _init():
        state[...] = jnp.zeros((1, _NRES), jnp.float32)

    s = state[...]
    part[0:1] = jax.lax.dot_general(s[:, 0:2048], wresT_ref[0:2048, :], _DN,
                                    preferred_element_type=jnp.float32)
    part[1:2] = jax.lax.dot_general(s[:, 2048:4096], wresT_ref[2048:4096, :], _DN,
                                    preferred_element_type=jnp.float32)
    new_state = jnp.tanh(u_ref[0] + (part[0:1] + part[1:2]))
    state[...] = new_state
    o_ref[0] = new_state


def kernel(X, W_in, W_res):
    X2 = X[:, :, 0]
    winT = W_in.T.astype(jnp.bfloat16)
    wresT = W_res.T.astype(jnp.bfloat16)

    U = pl.pallas_call(
        _proj_kernel,
        grid=(_SEQ,),
        in_specs=[
            pl.BlockSpec((1, 1, _NIN), lambda t: (t, 0, 0)),
            pl.BlockSpec((_NIN, _NRES), lambda t: (0, 0)),
        ],
        out_specs=pl.BlockSpec((1, 1, _NRES), lambda t: (t, 0, 0)),
        out_shape=jax.ShapeDtypeStruct((_SEQ, 1, _NRES), jnp.float32),
    )(X2[:, None, :], winT)

    out = pl.pallas_call(
        _esn_kernel,
        grid=(_SEQ,),
        compiler_params=pltpu.CompilerParams(
            vmem_limit_bytes=100 * 1024 * 1024),
        scratch_shapes=[
            pltpu.VMEM((1, _NRES), jnp.float32),
            pltpu.VMEM((8, _NRES), jnp.float32),
        ],
        in_specs=[
            pl.BlockSpec((1, 1, _NRES), lambda t: (t, 0, 0)),
            pl.BlockSpec((_NRES, _NRES), lambda t: (0, 0)),
        ],
        out_specs=pl.BlockSpec((1, 1, _NRES), lambda t: (t, 0, 0)),
        out_shape=jax.ShapeDtypeStruct((_SEQ, 1, _NRES), jnp.float32),
    )(U, wresT)
    return out[:, 0, :]


# speedup vs baseline: 1.0609x; 1.0609x over previous
import jax
import jax.numpy as jnp
from jax.experimental import pallas as pl
from jax.experimental.pallas import tpu as pltpu

_DN = (((1,), (0,)), ((), ()))
_SEQ = 512
_NRES = 4096
_NIN = 256


def _esn_kernel(x_ref, winT_ref, wresT_ref, o_ref, state, part):
    t = pl.program_id(0)

    @pl.when(t == 0)
    def _init():
        state[...] = jnp.zeros((1, _NRES), jnp.float32)

    s = state[...]
    x = x_ref[0]

    part[2:3] = jax.lax.dot_general(x, winT_ref[...], _DN,
                                    preferred_element_type=jnp.float32)
    part[0:1] = jax.lax.dot_general(s[:, 0:2048], wresT_ref[0:2048, :], _DN,
                                    preferred_element_type=jnp.float32)
    part[1:2] = jax.lax.dot_general(s[:, 2048:4096], wresT_ref[2048:4096, :], _DN,
                                    preferred_element_type=jnp.float32)
    new_state = jnp.tanh(part[2:3] + (part[0:1] + part[1:2]))
    state[...] = new_state
    o_ref[0] = new_state


def kernel(X, W_in, W_res):
    X2 = X[:, :, 0]
    winT = W_in.T.astype(jnp.bfloat16)
    wresT = W_res.T.astype(jnp.bfloat16)

    out = pl.pallas_call(
        _esn_kernel,
        grid=(_SEQ,),
        compiler_params=pltpu.CompilerParams(
            vmem_limit_bytes=100 * 1024 * 1024),
        scratch_shapes=[
            pltpu.VMEM((1, _NRES), jnp.float32),
            pltpu.VMEM((8, _NRES), jnp.float32),
        ],
        in_specs=[
            pl.BlockSpec((1, 1, _NIN), lambda t: (t, 0, 0)),
            pl.BlockSpec((_NIN, _NRES), lambda t: (0, 0)),
            pl.BlockSpec((_NRES, _NRES), lambda t: (0, 0)),
        ],
        out_specs=pl.BlockSpec((1, 1, _NRES), lambda t: (t, 0, 0)),
        out_shape=jax.ShapeDtypeStruct((_SEQ, 1, _NRES), jnp.float32),
    )(X2[:, None, :], winT, wresT)
    return out[:, 0, :]
